# Initial kernel scaffold; baseline (speedup 1.0000x reference)
import jax
import jax.numpy as jnp
from jax.experimental import pallas as pl


def kernel(x, edge_index, W1_1, b1_1, W2_1, b2_1, W1_2, b1_2, W2_2, b2_2, W1_3, b1_3, W2_3, b2_3):
    raise NotImplementedError("write your pallas kernel here")


# speedup vs baseline: 1.6589x; 1.6589x over previous
import jax
import jax.numpy as jnp
from jax.experimental import pallas as pl


def _sigmoid_pallas(h):
    def body(h_ref, o_ref):
        o_ref[...] = jax.nn.sigmoid(h_ref[...])

    return pl.pallas_call(
        body,
        out_shape=jax.ShapeDtypeStruct(h.shape, h.dtype),
    )(h)


def kernel(x, edge_index, W1_1, b1_1, W2_1, b2_1, W1_2, b1_2, W2_2, b2_2,
           W1_3, b1_3, W2_3, b2_3):
    N = x.shape[0]
    src = edge_index[0].astype(jnp.int32)
    dst = edge_index[1].astype(jnp.int32)

    def layer(h, W1, b1, W2, b2):
        y = h @ W1
        s = jax.ops.segment_max(jnp.take(y, src, axis=0), dst, num_segments=N)
        agg = s - y + b1
        agg = jnp.where(jnp.isfinite(agg), agg, 0.0)
        return agg @ W2 + b2

    h = jax.nn.relu(layer(x, W1_1, b1_1, W2_1, b2_1))
    h = jax.nn.relu(layer(h, W1_2, b1_2, W2_2, b2_2))
    h = layer(h, W1_3, b1_3, W2_3, b2_3)

    P = 100352
    hp = jnp.pad(h.reshape(-1), (0, P - N)).reshape(784, 128)
    out = _sigmoid_pallas(hp).reshape(-1)[:N].reshape(N, 1)
    return out


# speedup vs baseline: 5.5588x; 3.3509x over previous
import functools

import jax
import jax.numpy as jnp
from jax import lax
from jax.experimental import pallas as pl
from jax.experimental.pallas import tpu as pltpu
from jax.experimental.pallas import tpu_sc as plsc

_NC, _NS = 2, 16
_NW = _NC * _NS
_N = 100000
_E = 3200000
_NB = 64
_R = 1568
_NPAD = _NB * _R
_EW = _E // _NW
_EPAD = _E + 1024
_H = 64

_MESH = plsc.VectorSubcoreMesh(
    core_axis_name="c", subcore_axis_name="s", num_cores=_NC, num_subcores=_NS)


def _wid():
    return lax.axis_index("s") * _NC + lax.axis_index("c")


def _bucket(d):
    return ((d.astype(jnp.float32) + jnp.float32(0.5))
            * jnp.float32(1.0 / _R)).astype(jnp.int32)


_CH1 = 4000


@functools.partial(
    pl.kernel,
    out_type=jax.ShapeDtypeStruct((_NW, 8, 128), jnp.int32),
    mesh=_MESH,
    scratch_types=[pltpu.VMEM((_CH1,), jnp.int32),
                   pltpu.VMEM((8, 128), jnp.int32)],
    compiler_params=pltpu.CompilerParams(needs_layout_passes=False),
)
def _k1_hist(dst_hbm, counts_hbm, dst_v, hist):
    w = _wid()
    lanes = lax.iota(jnp.int32, 16)

    for r in range(8):
        for c in range(8):
            hist[r, pl.ds(c * 16, 16)] = jnp.zeros((16,), jnp.int32)

    def chunk(k, _):
        off = pl.multiple_of(w * _EW + k * _CH1, 8)
        pltpu.sync_copy(dst_hbm.at[pl.ds(off, _CH1)], dst_v)

        def vec(i, _):
            d = dst_v[pl.ds(i * 16, 16)]
            idx = _bucket(d) * 16 + lanes
            cur = plsc.load_gather(hist, [idx >> 7, idx & 127])
            plsc.store_scatter(hist, [idx >> 7, idx & 127], cur + 1)
            return 0
        lax.fori_loop(0, _CH1 // 16, vec, 0)
        return 0
    lax.fori_loop(0, _EW // _CH1, chunk, 0)
    pltpu.sync_copy(hist, counts_hbm.at[w])


_CH2 = 2048


@functools.partial(
    pl.kernel,
    out_type=(jax.ShapeDtypeStruct((_EPAD,), jnp.int32),
              jax.ShapeDtypeStruct((2 * _NB,), jnp.int32)),
    mesh=_MESH,
    scratch_types=[pltpu.VMEM((_NW, 8, 128), jnp.int32),
                   pltpu.VMEM((_NB * 16,), jnp.int32),
                   pltpu.VMEM((2 * _NB,), jnp.int32),
                   pltpu.VMEM((_CH2,), jnp.int32),
                   pltpu.VMEM((_CH2,), jnp.int32),
                   pltpu.VMEM((1, 128), jnp.int32),
                   pltpu.VMEM((128,), jnp.int32),
                   pltpu.SemaphoreType.DMA],
    compiler_params=pltpu.CompilerParams(needs_layout_passes=False),
)
def _k2_scatter(src_hbm, dst_hbm, counts_hbm, bedge_hbm, meta_hbm,
                counts_v, offs, meta_v, sv, dv, pos_st, pack_st, sem_a):
    w = _wid()
    lanes = lax.iota(jnp.int32, 16)
    pltpu.sync_copy(counts_hbm, counts_v)
    z16 = jnp.zeros((16,), jnp.int32)

    def bloop(b, T):
        def wloop(wi, c):
            s_below, my_pex, total_b = c
            v = counts_v[wi, b >> 3, pl.ds((b & 7) * 16, 16)]
            tot = jnp.sum(v)
            s_below = s_below + jnp.where(wi < w, tot, 0)
            my_pex = jnp.where(wi == w, plsc.cumsum(v) - v, my_pex)
            return (s_below, my_pex, total_b + tot)
        s_below, my_pex, total_b = lax.fori_loop(
            0, _NW, wloop, (jnp.int32(0), z16, jnp.int32(0)))
        offs[pl.ds(b * 16, 16)] = T + s_below + my_pex

        @pl.when(w == 0)
        def _():
            bb = jnp.broadcast_to(b, (16,)).astype(jnp.int32)
            plsc.store_scatter(meta_v, [bb],
                               jnp.broadcast_to(T, (16,)).astype(jnp.int32),
                               mask=lanes == 0)
            plsc.store_scatter(meta_v, [bb + _NB],
                               jnp.broadcast_to(total_b, (16,)).astype(jnp.int32),
                               mask=lanes == 0)
        return jnp.bitwise_and(T + total_b + 7, jnp.int32(-8))

    lax.fori_loop(0, _NB, bloop, jnp.int32(0))

    @pl.when(w == 0)
    def _():
        pltpu.sync_copy(meta_v, meta_hbm)

    def stage_vec(jsrc, i):
        s = sv[pl.ds(jsrc * 16, 16)]
        d = dv[pl.ds(jsrc * 16, 16)]
        bkt = _bucket(d)
        idx = bkt * 16 + lanes
        pos = plsc.load_gather(offs, [idx])
        plsc.store_scatter(offs, [idx], pos + 1)
        pos_st[0, pl.ds(i * 16, 16)] = pos
        pack_st[pl.ds(i * 16, 16)] = (s << 11) | (d - bkt * _R)

    def fire_group():
        pltpu.async_copy(pack_st, bedge_hbm.at[pos_st.at[0]], sem_a).wait()

    def do_chunk(base_e, nvec):
        base_e = pl.multiple_of(base_e, 8)
        n = nvec * 16
        pltpu.sync_copy(src_hbm.at[pl.ds(base_e, n)], sv.at[pl.ds(0, n)])
        pltpu.sync_copy(dst_hbm.at[pl.ds(base_e, n)], dv.at[pl.ds(0, n)])

        def group(g, _):
            def vec(i, _):
                stage_vec(g * 8 + i, i)
                return 0
            lax.fori_loop(0, 8, vec, 0)
            fire_group()
            return 0
        lax.fori_loop(0, nvec >> 3, group, 0)

    def chunk(k, _):
        do_chunk(w * _EW + k * _CH2, _CH2 // 16)
        return 0
    lax.fori_loop(0, 48, chunk, 0)
    tail = w * _EW + 48 * _CH2
    do_chunk(tail, 104)
    pltpu.sync_copy(src_hbm.at[pl.ds(pl.multiple_of(tail + 1664, 8), 32)],
                    sv.at[pl.ds(0, 32)])
    pltpu.sync_copy(dst_hbm.at[pl.ds(pl.multiple_of(tail + 1664, 8), 32)],
                    dv.at[pl.ds(0, 32)])
    for i in range(2):
        stage_vec(i, i)
    for i in range(2, 8):
        pos_st[0, pl.ds(i * 16, 16)] = jnp.full((16,), _EPAD - 128 + i * 16,
                                                jnp.int32) + lanes
        pack_st[pl.ds(i * 16, 16)] = jnp.zeros((16,), jnp.int32)
    fire_group()


_CH3 = 512


@functools.partial(
    pl.kernel,
    out_type=jax.ShapeDtypeStruct((_NPAD * _H,), jnp.float32),
    mesh=_MESH,
    scratch_types=[pltpu.VMEM((_R * _H,), jnp.float32),
                   pltpu.VMEM((_CH3,), jnp.int32),
                   pltpu.VMEM((_CH3,), jnp.int32),
                   pltpu.VMEM((_CH3 + 16,), jnp.int32),
                   pltpu.VMEM((128, 128), jnp.float32),
                   pltpu.VMEM((2 * _NB + 16,), jnp.int32),
                   pltpu.SemaphoreType.DMA],
    compiler_params=pltpu.CompilerParams(needs_layout_passes=False),
)
def _k3_segmax(y_hbm, meta_hbm, bedge_hbm, s_hbm,
               acc, ebuf, esrc, edst, rows, meta_v, sem):
    w = _wid()
    lanes = lax.iota(jnp.int32, 16)
    pltpu.sync_copy(meta_hbm, meta_v.at[pl.ds(0, 2 * _NB)])
    ninf = jnp.full((16,), -jnp.inf, jnp.float32)

    for j in range(2):
        b = w * 2 + j
        base = pl.multiple_of(meta_v[pl.ds(b, 16)][0], 8)
        cnt = meta_v[pl.ds(_NB + b, 16)][0]

        def zr(r, _):
            acc[pl.ds(r * 16, 16)] = ninf
            return 0
        lax.fori_loop(0, _R * _H // 16, zr, 0)

        def chunk(k, _):
            eoff = pl.multiple_of(base + k * _CH3, 8)
            pltpu.sync_copy(bedge_hbm.at[pl.ds(eoff, _CH3)], ebuf)

            def sanitize(i, _):
                valid = (k * _CH3 + i * 16 + lanes) < cnt
                p = ebuf[pl.ds(i * 16, 16)]
                esrc[pl.ds(i * 16, 16)] = jnp.where(valid, p >> 11, 0)
                edst[pl.ds(i * 16, 16)] = jnp.where(valid, p & 2047, 0)
                return 0
            lax.fori_loop(0, _CH3 // 16, sanitize, 0)

            for g in range(_CH3 // 128):
                pltpu.async_copy(
                    y_hbm.at[esrc.at[pl.ds(g * 128, 128)]], rows, sem).wait()

                def edge(e, _):
                    eg = k * _CH3 + g * 128 + e

                    @pl.when(eg < cnt)
                    def _():
                        dloc = edst[pl.ds(g * 128 + e, 16)][0]
                        rbase = dloc * _H
                        for c in range(4):
                            rv = rows[e, pl.ds(c * 16, 16)]
                            av = acc[pl.ds(rbase + c * 16, 16)]
                            acc[pl.ds(rbase + c * 16, 16)] = jnp.maximum(av, rv)
                    return 0
                lax.fori_loop(0, 128, edge, 0)
            return 0
        nch = (cnt + _CH3 - 1) >> 9
        lax.fori_loop(0, nch, chunk, 0)
        pltpu.sync_copy(acc, s_hbm.at[pl.ds(b * (_R * _H), _R * _H)])


_BR = 1024
_GRID = (_NPAD + _BR - 1) // _BR


def _tc_y0(xp, W1p):
    def body(x_ref, w_ref, o_ref):
        o_ref[...] = jnp.dot(x_ref[...], w_ref[...],
                             preferred_element_type=jnp.float32)
    return pl.pallas_call(
        body,
        grid=(_GRID,),
        in_specs=[pl.BlockSpec((_BR, 4), lambda i: (i, 0)),
                  pl.BlockSpec((4, 128), lambda i: (0, 0))],
        out_specs=pl.BlockSpec((_BR, 128), lambda i: (i, 0)),
        out_shape=jax.ShapeDtypeStruct((_NPAD, 128), jnp.float32),
    )(xp, W1p)


def _tc_mid(s, y, b1, W2, b2, W1n):
    f_out = W2.shape[1]

    def body(s_ref, y_ref, b1_ref, w2_ref, b2_ref, w1n_ref, o_ref):
        agg = s_ref[...] - y_ref[...][:, :_H] + b1_ref[...]
        agg = jnp.where(jnp.isfinite(agg), agg, 0.0)
        z = jnp.dot(agg, w2_ref[...],
                    preferred_element_type=jnp.float32) + b2_ref[...]
        h = jnp.maximum(z, 0.0)
        o_ref[...] = jnp.dot(h, w1n_ref[...],
                             preferred_element_type=jnp.float32)
    return pl.pallas_call(
        body,
        grid=(_GRID,),
        in_specs=[pl.BlockSpec((_BR, _H), lambda i: (i, 0)),
                  pl.BlockSpec((_BR, 128), lambda i: (i, 0)),
                  pl.BlockSpec((1, _H), lambda i: (0, 0)),
                  pl.BlockSpec((_H, f_out), lambda i: (0, 0)),
                  pl.BlockSpec((1, f_out), lambda i: (0, 0)),
                  pl.BlockSpec((f_out, 128), lambda i: (0, 0))],
        out_specs=pl.BlockSpec((_BR, 128), lambda i: (i, 0)),
        out_shape=jax.ShapeDtypeStruct((_NPAD, 128), jnp.float32),
    )(s, y, b1, W2, b2, W1n)


def _tc_final(s, y, b1, W2, b2):
    def body(s_ref, y_ref, b1_ref, w2_ref, b2_ref, o_ref):
        agg = s_ref[...] - y_ref[...][:, :_H] + b1_ref[...]
        agg = jnp.where(jnp.isfinite(agg), agg, 0.0)
        z = jnp.dot(agg, w2_ref[...],
                    preferred_element_type=jnp.float32) + b2_ref[...]
        o_ref[...] = jax.nn.sigmoid(z)
    return pl.pallas_call(
        body,
        grid=(_GRID,),
        in_specs=[pl.BlockSpec((_BR, _H), lambda i: (i, 0)),
                  pl.BlockSpec((_BR, 128), lambda i: (i, 0)),
                  pl.BlockSpec((1, _H), lambda i: (0, 0)),
                  pl.BlockSpec((_H, 1), lambda i: (0, 0)),
                  pl.BlockSpec((1, 1), lambda i: (0, 0))],
        out_specs=pl.BlockSpec((_BR, 1), lambda i: (i, 0)),
        out_shape=jax.ShapeDtypeStruct((_NPAD, 1), jnp.float32),
    )(s, y, b1, W2, b2)


def kernel(x, edge_index, W1_1, b1_1, W2_1, b2_1, W1_2, b1_2, W2_2, b2_2,
           W1_3, b1_3, W2_3, b2_3):
    src = edge_index[0].astype(jnp.int32)
    dst = edge_index[1].astype(jnp.int32)

    counts = _k1_hist(dst)
    bedge, meta = _k2_scatter(src, dst, counts)

    xp = jnp.pad(x, ((0, _NPAD - _N), (0, 1)))
    W1_1p = jnp.pad(W1_1, ((0, 1), (0, 64)))
    W1_2p = jnp.pad(W1_2, ((0, 0), (0, 64)))
    W1_3p = jnp.pad(W1_3, ((0, 0), (0, 64)))
    y = _tc_y0(xp, W1_1p)

    s = _k3_segmax(y, meta, bedge).reshape(_NPAD, _H)
    y = _tc_mid(s, y, b1_1.reshape(1, -1), W2_1, b2_1.reshape(1, -1), W1_2p)
    s = _k3_segmax(y, meta, bedge).reshape(_NPAD, _H)
    y = _tc_mid(s, y, b1_2.reshape(1, -1), W2_2, b2_2.reshape(1, -1), W1_3p)
    s = _k3_segmax(y, meta, bedge).reshape(_NPAD, _H)
    out = _tc_final(s, y, b1_3.reshape(1, -1), W2_3, b2_3.reshape(1, -1))
    return out[:_N]


# speedup vs baseline: 6.3371x; 1.1400x over previous
import functools

import jax
import jax.numpy as jnp
from jax import lax
from jax.experimental import pallas as pl
from jax.experimental.pallas import tpu as pltpu
from jax.experimental.pallas import tpu_sc as plsc

_NC, _NS = 2, 16
_NW = _NC * _NS
_N = 100000
_E = 3200000
_NB = 64
_R = 1568
_NPAD = _NB * _R
_EW = _E // _NW
_EPAD = _E + 1024
_H = 64

_MESH = plsc.VectorSubcoreMesh(
    core_axis_name="c", subcore_axis_name="s", num_cores=_NC, num_subcores=_NS)


def _wid():
    return lax.axis_index("s") * _NC + lax.axis_index("c")


def _bucket(d):
    return ((d.astype(jnp.float32) + jnp.float32(0.5))
            * jnp.float32(1.0 / _R)).astype(jnp.int32)


_CH1 = 4000


@functools.partial(
    pl.kernel,
    out_type=jax.ShapeDtypeStruct((_NW, 8, 128), jnp.int32),
    mesh=_MESH,
    scratch_types=[pltpu.VMEM((_CH1,), jnp.int32),
                   pltpu.VMEM((8, 128), jnp.int32)],
    compiler_params=pltpu.CompilerParams(needs_layout_passes=False),
)
def _k1_hist(dst_hbm, counts_hbm, dst_v, hist):
    w = _wid()
    lanes = lax.iota(jnp.int32, 16)

    for r in range(8):
        for c in range(8):
            hist[r, pl.ds(c * 16, 16)] = jnp.zeros((16,), jnp.int32)

    def chunk(k, _):
        off = pl.multiple_of(w * _EW + k * _CH1, 8)
        pltpu.sync_copy(dst_hbm.at[pl.ds(off, _CH1)], dst_v)

        def vec(i, _):
            d = dst_v[pl.ds(i * 16, 16)]
            idx = _bucket(d) * 16 + lanes
            cur = plsc.load_gather(hist, [idx >> 7, idx & 127])
            plsc.store_scatter(hist, [idx >> 7, idx & 127], cur + 1)
            return 0
        lax.fori_loop(0, _CH1 // 16, vec, 0)
        return 0
    lax.fori_loop(0, _EW // _CH1, chunk, 0)
    pltpu.sync_copy(hist, counts_hbm.at[w])


_CH2 = 2048


@functools.partial(
    pl.kernel,
    out_type=(jax.ShapeDtypeStruct((_EPAD,), jnp.int32),
              jax.ShapeDtypeStruct((2 * _NB,), jnp.int32)),
    mesh=_MESH,
    scratch_types=[pltpu.VMEM((_NW, 8, 128), jnp.int32),
                   pltpu.VMEM((_NB * 16,), jnp.int32),
                   pltpu.VMEM((2 * _NB,), jnp.int32),
                   pltpu.VMEM((_CH2,), jnp.int32),
                   pltpu.VMEM((_CH2,), jnp.int32),
                   pltpu.VMEM((2, 128), jnp.int32),
                   pltpu.VMEM((2, 128), jnp.int32),
                   pltpu.SemaphoreType.DMA,
                   pltpu.SemaphoreType.DMA],
    compiler_params=pltpu.CompilerParams(needs_layout_passes=False),
)
def _k2_scatter(src_hbm, dst_hbm, counts_hbm, bedge_hbm, meta_hbm,
                counts_v, offs, meta_v, sv, dv, pos_st, pack_st,
                sem_a, sem_b):
    w = _wid()
    lanes = lax.iota(jnp.int32, 16)
    pltpu.sync_copy(counts_hbm, counts_v)
    z16 = jnp.zeros((16,), jnp.int32)

    def bloop(b, T):
        def wloop(wi, c):
            s_below, my_pex, total_b = c
            v = counts_v[wi, b >> 3, pl.ds((b & 7) * 16, 16)]
            tot = jnp.sum(v)
            s_below = s_below + jnp.where(wi < w, tot, 0)
            my_pex = jnp.where(wi == w, plsc.cumsum(v) - v, my_pex)
            return (s_below, my_pex, total_b + tot)
        s_below, my_pex, total_b = lax.fori_loop(
            0, _NW, wloop, (jnp.int32(0), z16, jnp.int32(0)))
        offs[pl.ds(b * 16, 16)] = T + s_below + my_pex

        @pl.when(w == 0)
        def _():
            bb = jnp.broadcast_to(b, (16,)).astype(jnp.int32)
            plsc.store_scatter(meta_v, [bb],
                               jnp.broadcast_to(T, (16,)).astype(jnp.int32),
                               mask=lanes == 0)
            plsc.store_scatter(meta_v, [bb + _NB],
                               jnp.broadcast_to(total_b, (16,)).astype(jnp.int32),
                               mask=lanes == 0)
        return jnp.bitwise_and(T + total_b + 7, jnp.int32(-8))

    lax.fori_loop(0, _NB, bloop, jnp.int32(0))

    @pl.when(w == 0)
    def _():
        pltpu.sync_copy(meta_v, meta_hbm)

    sems = (sem_a, sem_b)

    def stage_vec(jsrc, par, i):
        s = sv[pl.ds(jsrc * 16, 16)]
        d = dv[pl.ds(jsrc * 16, 16)]
        bkt = _bucket(d)
        idx = bkt * 16 + lanes
        pos = plsc.load_gather(offs, [idx])
        plsc.store_scatter(offs, [idx], pos + 1)
        pos_st[par, pl.ds(i * 16, 16)] = pos
        pack_st[par, pl.ds(i * 16, 16)] = (s << 11) | (d - bkt * _R)

    def fire_group(par):
        return pltpu.async_copy(pack_st.at[par], bedge_hbm.at[pos_st.at[par]],
                                sems[par])

    def do_chunk(base_e, nvec):
        base_e = pl.multiple_of(base_e, 8)
        n = nvec * 16
        pltpu.sync_copy(src_hbm.at[pl.ds(base_e, n)], sv.at[pl.ds(0, n)])
        pltpu.sync_copy(dst_hbm.at[pl.ds(base_e, n)], dv.at[pl.ds(0, n)])
        descs = [None, None]
        for g in range(nvec >> 3):
            par = g & 1
            if descs[par] is not None:
                descs[par].wait()

            def vec(i, _, g=g, par=par):
                stage_vec(g * 8 + i, par, i)
                return 0
            lax.fori_loop(0, 8, vec, 0)
            descs[par] = fire_group(par)
        for d_ in descs:
            if d_ is not None:
                d_.wait()

    def chunk(k, _):
        do_chunk(w * _EW + k * _CH2, _CH2 // 16)
        return 0
    lax.fori_loop(0, 48, chunk, 0)
    tail = w * _EW + 48 * _CH2
    do_chunk(tail, 104)
    pltpu.sync_copy(src_hbm.at[pl.ds(pl.multiple_of(tail + 1664, 8), 32)],
                    sv.at[pl.ds(0, 32)])
    pltpu.sync_copy(dst_hbm.at[pl.ds(pl.multiple_of(tail + 1664, 8), 32)],
                    dv.at[pl.ds(0, 32)])
    for i in range(2):
        stage_vec(i, 0, i)
    for i in range(2, 8):
        pos_st[0, pl.ds(i * 16, 16)] = jnp.full((16,), _EPAD - 128 + i * 16,
                                                jnp.int32) + lanes
        pack_st[0, pl.ds(i * 16, 16)] = jnp.zeros((16,), jnp.int32)
    fire_group(0).wait()


_CH3 = 512


@functools.partial(
    pl.kernel,
    out_type=jax.ShapeDtypeStruct((_NPAD * _H,), jnp.float32),
    mesh=_MESH,
    scratch_types=[pltpu.VMEM((_R * _H,), jnp.float32),
                   pltpu.VMEM((_CH3,), jnp.int32),
                   pltpu.VMEM((_CH3,), jnp.int32),
                   pltpu.VMEM((_CH3 + 16,), jnp.int32),
                   pltpu.VMEM((2, 64, 128), jnp.float32),
                   pltpu.VMEM((2 * _NB + 16,), jnp.int32),
                   pltpu.SemaphoreType.DMA,
                   pltpu.SemaphoreType.DMA],
    compiler_params=pltpu.CompilerParams(needs_layout_passes=False),
)
def _k3_segmax(y_hbm, meta_hbm, bedge_hbm, s_hbm,
               acc, ebuf, esrc, edst, rows, meta_v, sem0, sem1):
    w = _wid()
    lanes = lax.iota(jnp.int32, 16)
    sems = (sem0, sem1)
    pltpu.sync_copy(meta_hbm, meta_v.at[pl.ds(0, 2 * _NB)])
    ninf = jnp.full((16,), -jnp.inf, jnp.float32)
    ngrp = _CH3 // 64

    for j in range(2):
        b = w * 2 + j
        base = pl.multiple_of(meta_v[pl.ds(b, 16)][0], 8)
        cnt = meta_v[pl.ds(_NB + b, 16)][0]

        def zr(r, _):
            acc[pl.ds(r * 16, 16)] = ninf
            return 0
        lax.fori_loop(0, _R * _H // 16, zr, 0)

        def chunk(k, _):
            eoff = pl.multiple_of(base + k * _CH3, 8)
            pltpu.sync_copy(bedge_hbm.at[pl.ds(eoff, _CH3)], ebuf)

            def sanitize(i, _):
                valid = (k * _CH3 + i * 16 + lanes) < cnt
                p = ebuf[pl.ds(i * 16, 16)]
                esrc[pl.ds(i * 16, 16)] = jnp.where(valid, p >> 11, 0)
                edst[pl.ds(i * 16, 16)] = jnp.where(valid, p & 2047, 0)
                return 0
            lax.fori_loop(0, _CH3 // 16, sanitize, 0)

            descs = [None, None]
            descs[0] = pltpu.async_copy(
                y_hbm.at[esrc.at[pl.ds(0, 64)]], rows.at[0], sems[0])
            for g in range(ngrp):
                if g + 1 < ngrp:
                    nb = (g + 1) & 1
                    descs[nb] = pltpu.async_copy(
                        y_hbm.at[esrc.at[pl.ds((g + 1) * 64, 64)]],
                        rows.at[nb], sems[nb])
                cur = g & 1
                descs[cur].wait()

                def edge(e, _):
                    eg = k * _CH3 + g * 64 + e

                    @pl.when(eg < cnt)
                    def _():
                        dloc = edst[pl.ds(g * 64 + e, 16)][0]
                        rbase = dloc * _H
                        for c in range(4):
                            rv = rows[cur, e, pl.ds(c * 16, 16)]
                            av = acc[pl.ds(rbase + c * 16, 16)]
                            acc[pl.ds(rbase + c * 16, 16)] = jnp.maximum(av, rv)
                    return 0
                lax.fori_loop(0, 64, edge, 0)
            return 0
        nch = (cnt + _CH3 - 1) >> 9
        lax.fori_loop(0, nch, chunk, 0)
        pltpu.sync_copy(acc, s_hbm.at[pl.ds(b * (_R * _H), _R * _H)])


_BR = 1024
_GRID = (_NPAD + _BR - 1) // _BR


def _tc_y0(xp, W1p):
    def body(x_ref, w_ref, o_ref):
        o_ref[...] = jnp.dot(x_ref[...], w_ref[...],
                             preferred_element_type=jnp.float32)
    return pl.pallas_call(
        body,
        grid=(_GRID,),
        in_specs=[pl.BlockSpec((_BR, 4), lambda i: (i, 0)),
                  pl.BlockSpec((4, 128), lambda i: (0, 0))],
        out_specs=pl.BlockSpec((_BR, 128), lambda i: (i, 0)),
        out_shape=jax.ShapeDtypeStruct((_NPAD, 128), jnp.float32),
    )(xp, W1p)


def _tc_mid(s, y, b1, W2, b2, W1n):
    f_out = W2.shape[1]

    def body(s_ref, y_ref, b1_ref, w2_ref, b2_ref, w1n_ref, o_ref):
        agg = s_ref[...] - y_ref[...][:, :_H] + b1_ref[...]
        agg = jnp.where(jnp.isfinite(agg), agg, 0.0)
        z = jnp.dot(agg, w2_ref[...],
                    preferred_element_type=jnp.float32) + b2_ref[...]
        h = jnp.maximum(z, 0.0)
        o_ref[...] = jnp.dot(h, w1n_ref[...],
                             preferred_element_type=jnp.float32)
    return pl.pallas_call(
        body,
        grid=(_GRID,),
        in_specs=[pl.BlockSpec((_BR, _H), lambda i: (i, 0)),
                  pl.BlockSpec((_BR, 128), lambda i: (i, 0)),
                  pl.BlockSpec((1, _H), lambda i: (0, 0)),
                  pl.BlockSpec((_H, f_out), lambda i: (0, 0)),
                  pl.BlockSpec((1, f_out), lambda i: (0, 0)),
                  pl.BlockSpec((f_out, 128), lambda i: (0, 0))],
        out_specs=pl.BlockSpec((_BR, 128), lambda i: (i, 0)),
        out_shape=jax.ShapeDtypeStruct((_NPAD, 128), jnp.float32),
    )(s, y, b1, W2, b2, W1n)


def _tc_final(s, y, b1, W2, b2):
    def body(s_ref, y_ref, b1_ref, w2_ref, b2_ref, o_ref):
        agg = s_ref[...] - y_ref[...][:, :_H] + b1_ref[...]
        agg = jnp.where(jnp.isfinite(agg), agg, 0.0)
        z = jnp.dot(agg, w2_ref[...],
                    preferred_element_type=jnp.float32) + b2_ref[...]
        o_ref[...] = jax.nn.sigmoid(z)
    return pl.pallas_call(
        body,
        grid=(_GRID,),
        in_specs=[pl.BlockSpec((_BR, _H), lambda i: (i, 0)),
                  pl.BlockSpec((_BR, 128), lambda i: (i, 0)),
                  pl.BlockSpec((1, _H), lambda i: (0, 0)),
                  pl.BlockSpec((_H, 1), lambda i: (0, 0)),
                  pl.BlockSpec((1, 1), lambda i: (0, 0))],
        out_specs=pl.BlockSpec((_BR, 1), lambda i: (i, 0)),
        out_shape=jax.ShapeDtypeStruct((_NPAD, 1), jnp.float32),
    )(s, y, b1, W2, b2)


def kernel(x, edge_index, W1_1, b1_1, W2_1, b2_1, W1_2, b1_2, W2_2, b2_2,
           W1_3, b1_3, W2_3, b2_3):
    src = edge_index[0].astype(jnp.int32)
    dst = edge_index[1].astype(jnp.int32)

    counts = _k1_hist(dst)
    bedge, meta = _k2_scatter(src, dst, counts)

    xp = jnp.pad(x, ((0, _NPAD - _N), (0, 1)))
    W1_1p = jnp.pad(W1_1, ((0, 1), (0, 64)))
    W1_2p = jnp.pad(W1_2, ((0, 0), (0, 64)))
    W1_3p = jnp.pad(W1_3, ((0, 0), (0, 64)))
    y = _tc_y0(xp, W1_1p)

    s = _k3_segmax(y, meta, bedge).reshape(_NPAD, _H)
    y = _tc_mid(s, y, b1_1.reshape(1, -1), W2_1, b2_1.reshape(1, -1), W1_2p)
    s = _k3_segmax(y, meta, bedge).reshape(_NPAD, _H)
    y = _tc_mid(s, y, b1_2.reshape(1, -1), W2_2, b2_2.reshape(1, -1), W1_3p)
    s = _k3_segmax(y, meta, bedge).reshape(_NPAD, _H)
    out = _tc_final(s, y, b1_3.reshape(1, -1), W2_3, b2_3.reshape(1, -1))
    return out[:_N]


# speedup vs baseline: 6.3571x; 1.0032x over previous
import functools

import jax
import jax.numpy as jnp
from jax import lax
from jax.experimental import pallas as pl
from jax.experimental.pallas import tpu as pltpu
from jax.experimental.pallas import tpu_sc as plsc

_NC, _NS = 2, 16
_NW = _NC * _NS
_N = 100000
_E = 3200000
_NB = 64
_R = 1568
_NPAD = _NB * _R
_EW = _E // _NW
_EPAD = _E + 1024
_H = 64

_MESH = plsc.VectorSubcoreMesh(
    core_axis_name="c", subcore_axis_name="s", num_cores=_NC, num_subcores=_NS)


def _wid():
    return lax.axis_index("s") * _NC + lax.axis_index("c")


def _bucket(d):
    return ((d.astype(jnp.float32) + jnp.float32(0.5))
            * jnp.float32(1.0 / _R)).astype(jnp.int32)


_CH1 = 4000


@functools.partial(
    pl.kernel,
    out_type=jax.ShapeDtypeStruct((_NW, 8, 128), jnp.int32),
    mesh=_MESH,
    scratch_types=[pltpu.VMEM((_CH1,), jnp.int32),
                   pltpu.VMEM((8, 128), jnp.int32)],
    compiler_params=pltpu.CompilerParams(needs_layout_passes=False),
)
def _k1_hist(dst_hbm, counts_hbm, dst_v, hist):
    w = _wid()
    lanes = lax.iota(jnp.int32, 16)

    for r in range(8):
        for c in range(8):
            hist[r, pl.ds(c * 16, 16)] = jnp.zeros((16,), jnp.int32)

    def chunk(k, _):
        off = pl.multiple_of(w * _EW + k * _CH1, 8)
        pltpu.sync_copy(dst_hbm.at[pl.ds(off, _CH1)], dst_v)

        def vec(i, _):
            d = dst_v[pl.ds(i * 16, 16)]
            idx = _bucket(d) * 16 + lanes
            cur = plsc.load_gather(hist, [idx >> 7, idx & 127])
            plsc.store_scatter(hist, [idx >> 7, idx & 127], cur + 1)
            return 0
        lax.fori_loop(0, _CH1 // 16, vec, 0)
        return 0
    lax.fori_loop(0, _EW // _CH1, chunk, 0)
    pltpu.sync_copy(hist, counts_hbm.at[w])


_CH2 = 2048


@functools.partial(
    pl.kernel,
    out_type=(jax.ShapeDtypeStruct((_EPAD,), jnp.int32),
              jax.ShapeDtypeStruct((2 * _NB,), jnp.int32)),
    mesh=_MESH,
    scratch_types=[pltpu.VMEM((_NW, 8, 128), jnp.int32),
                   pltpu.VMEM((_NB * 16,), jnp.int32),
                   pltpu.VMEM((2 * _NB,), jnp.int32),
                   pltpu.VMEM((_CH2,), jnp.int32),
                   pltpu.VMEM((_CH2,), jnp.int32),
                   pltpu.VMEM((2, 128), jnp.int32),
                   pltpu.VMEM((2, 128), jnp.int32),
                   pltpu.SemaphoreType.DMA,
                   pltpu.SemaphoreType.DMA],
    compiler_params=pltpu.CompilerParams(needs_layout_passes=False),
)
def _k2_scatter(src_hbm, dst_hbm, counts_hbm, bedge_hbm, meta_hbm,
                counts_v, offs, meta_v, sv, dv, pos_st, pack_st,
                sem_a, sem_b):
    w = _wid()
    lanes = lax.iota(jnp.int32, 16)
    pltpu.sync_copy(counts_hbm, counts_v)
    z16 = jnp.zeros((16,), jnp.int32)

    def bloop(b, T):
        def wloop(wi, c):
            s_below, my_pex, total_b = c
            v = counts_v[wi, b >> 3, pl.ds((b & 7) * 16, 16)]
            tot = jnp.sum(v)
            s_below = s_below + jnp.where(wi < w, tot, 0)
            my_pex = jnp.where(wi == w, plsc.cumsum(v) - v, my_pex)
            return (s_below, my_pex, total_b + tot)
        s_below, my_pex, total_b = lax.fori_loop(
            0, _NW, wloop, (jnp.int32(0), z16, jnp.int32(0)))
        offs[pl.ds(b * 16, 16)] = T + s_below + my_pex

        @pl.when(w == 0)
        def _():
            bb = jnp.broadcast_to(b, (16,)).astype(jnp.int32)
            plsc.store_scatter(meta_v, [bb],
                               jnp.broadcast_to(T, (16,)).astype(jnp.int32),
                               mask=lanes == 0)
            plsc.store_scatter(meta_v, [bb + _NB],
                               jnp.broadcast_to(total_b, (16,)).astype(jnp.int32),
                               mask=lanes == 0)
        return jnp.bitwise_and(T + total_b + 7, jnp.int32(-8))

    lax.fori_loop(0, _NB, bloop, jnp.int32(0))

    @pl.when(w == 0)
    def _():
        pltpu.sync_copy(meta_v, meta_hbm)

    sems = (sem_a, sem_b)

    def stage_vec(jsrc, par, i):
        s = sv[pl.ds(jsrc * 16, 16)]
        d = dv[pl.ds(jsrc * 16, 16)]
        bkt = _bucket(d)
        idx = bkt * 16 + lanes
        pos = plsc.load_gather(offs, [idx])
        plsc.store_scatter(offs, [idx], pos + 1)
        pos_st[par, pl.ds(i * 16, 16)] = pos
        pack_st[par, pl.ds(i * 16, 16)] = (s << 11) | (d - bkt * _R)

    def fire_group(par):
        return pltpu.async_copy(pack_st.at[par], bedge_hbm.at[pos_st.at[par]],
                                sems[par])

    def do_chunk(base_e, nvec):
        base_e = pl.multiple_of(base_e, 8)
        n = nvec * 16
        pltpu.sync_copy(src_hbm.at[pl.ds(base_e, n)], sv.at[pl.ds(0, n)])
        pltpu.sync_copy(dst_hbm.at[pl.ds(base_e, n)], dv.at[pl.ds(0, n)])
        descs = [None, None]
        for g in range(nvec >> 3):
            par = g & 1
            if descs[par] is not None:
                descs[par].wait()

            def vec(i, _, g=g, par=par):
                stage_vec(g * 8 + i, par, i)
                return 0
            lax.fori_loop(0, 8, vec, 0)
            descs[par] = fire_group(par)
        for d_ in descs:
            if d_ is not None:
                d_.wait()

    def chunk(k, _):
        do_chunk(w * _EW + k * _CH2, _CH2 // 16)
        return 0
    lax.fori_loop(0, 48, chunk, 0)
    tail = w * _EW + 48 * _CH2
    do_chunk(tail, 104)
    pltpu.sync_copy(src_hbm.at[pl.ds(pl.multiple_of(tail + 1664, 8), 32)],
                    sv.at[pl.ds(0, 32)])
    pltpu.sync_copy(dst_hbm.at[pl.ds(pl.multiple_of(tail + 1664, 8), 32)],
                    dv.at[pl.ds(0, 32)])
    for i in range(2):
        stage_vec(i, 0, i)
    for i in range(2, 8):
        pos_st[0, pl.ds(i * 16, 16)] = jnp.full((16,), _EPAD - 128 + i * 16,
                                                jnp.int32) + lanes
        pack_st[0, pl.ds(i * 16, 16)] = jnp.zeros((16,), jnp.int32)
    fire_group(0).wait()


_CH3 = 512


@functools.partial(
    pl.kernel,
    out_type=jax.ShapeDtypeStruct((_NPAD * _H,), jnp.float32),
    mesh=_MESH,
    scratch_types=[pltpu.VMEM(((_R + 1) * _H,), jnp.float32),
                   pltpu.VMEM((_CH3,), jnp.int32),
                   pltpu.VMEM((_CH3,), jnp.int32),
                   pltpu.VMEM((_CH3 + 16,), jnp.int32),
                   pltpu.VMEM((3, 64, 128), jnp.float32),
                   pltpu.VMEM((2 * _NB + 16,), jnp.int32),
                   pltpu.SemaphoreType.DMA,
                   pltpu.SemaphoreType.DMA,
                   pltpu.SemaphoreType.DMA],
    compiler_params=pltpu.CompilerParams(needs_layout_passes=False),
)
def _k3_segmax(y_hbm, meta_hbm, bedge_hbm, s_hbm,
               acc, ebuf, esrc, edst, rows, meta_v, sem0, sem1, sem2):
    w = _wid()
    lanes = lax.iota(jnp.int32, 16)
    sems = (sem0, sem1, sem2)
    pltpu.sync_copy(meta_hbm, meta_v.at[pl.ds(0, 2 * _NB)])
    ninf = jnp.full((16,), -jnp.inf, jnp.float32)
    ngrp = _CH3 // 64

    for j in range(2):
        b = w * 2 + j
        base = pl.multiple_of(meta_v[pl.ds(b, 16)][0], 8)
        cnt = meta_v[pl.ds(_NB + b, 16)][0]

        def zr(r, _):
            acc[pl.ds(r * 16, 16)] = ninf
            return 0
        lax.fori_loop(0, (_R + 1) * _H // 16, zr, 0)

        def chunk(k, _):
            eoff = pl.multiple_of(base + k * _CH3, 8)
            pltpu.sync_copy(bedge_hbm.at[pl.ds(eoff, _CH3)], ebuf)

            def sanitize(i, _):
                valid = (k * _CH3 + i * 16 + lanes) < cnt
                p = ebuf[pl.ds(i * 16, 16)]
                esrc[pl.ds(i * 16, 16)] = jnp.where(valid, p >> 11, 0)
                edst[pl.ds(i * 16, 16)] = jnp.where(valid, p & 2047, _R)
                return 0
            lax.fori_loop(0, _CH3 // 16, sanitize, 0)

            def issue(g):
                return pltpu.async_copy(
                    y_hbm.at[esrc.at[pl.ds(g * 64, 64)]],
                    rows.at[g % 3], sems[g % 3])

            descs = [None, None, None]
            descs[0] = issue(0)
            descs[1] = issue(1)
            for g in range(ngrp):
                if g + 2 < ngrp:
                    descs[(g + 2) % 3] = issue(g + 2)
                bi = g % 3
                descs[bi].wait()

                def edge(e2, _, g=g, bi=bi):
                    for u in range(2):
                        e = e2 * 2 + u
                        dloc = edst[pl.ds(g * 64 + e, 16)][0]
                        rbase = dloc * _H
                        for c in range(4):
                            rv = rows[bi, e, pl.ds(c * 16, 16)]
                            av = acc[pl.ds(rbase + c * 16, 16)]
                            acc[pl.ds(rbase + c * 16, 16)] = \
                                jnp.maximum(av, rv)
                    return 0
                lax.fori_loop(0, 32, edge, 0)
            return 0
        nch = (cnt + _CH3 - 1) >> 9
        lax.fori_loop(0, nch, chunk, 0)
        pltpu.sync_copy(acc.at[pl.ds(0, _R * _H)],
                        s_hbm.at[pl.ds(b * (_R * _H), _R * _H)])


_BR = 1024
_GRID = (_NPAD + _BR - 1) // _BR


def _tc_y0(xp, W1p):
    def body(x_ref, w_ref, o_ref):
        o_ref[...] = jnp.dot(x_ref[...], w_ref[...],
                             preferred_element_type=jnp.float32)
    return pl.pallas_call(
        body,
        grid=(_GRID,),
        in_specs=[pl.BlockSpec((_BR, 4), lambda i: (i, 0)),
                  pl.BlockSpec((4, 128), lambda i: (0, 0))],
        out_specs=pl.BlockSpec((_BR, 128), lambda i: (i, 0)),
        out_shape=jax.ShapeDtypeStruct((_NPAD, 128), jnp.float32),
    )(xp, W1p)


def _tc_mid(s, y, b1, W2, b2, W1n):
    f_out = W2.shape[1]

    def body(s_ref, y_ref, b1_ref, w2_ref, b2_ref, w1n_ref, o_ref):
        agg = s_ref[...] - y_ref[...][:, :_H] + b1_ref[...]
        agg = jnp.where(jnp.isfinite(agg), agg, 0.0)
        z = jnp.dot(agg, w2_ref[...],
                    preferred_element_type=jnp.float32) + b2_ref[...]
        h = jnp.maximum(z, 0.0)
        o_ref[...] = jnp.dot(h, w1n_ref[...],
                             preferred_element_type=jnp.float32)
    return pl.pallas_call(
        body,
        grid=(_GRID,),
        in_specs=[pl.BlockSpec((_BR, _H), lambda i: (i, 0)),
                  pl.BlockSpec((_BR, 128), lambda i: (i, 0)),
                  pl.BlockSpec((1, _H), lambda i: (0, 0)),
                  pl.BlockSpec((_H, f_out), lambda i: (0, 0)),
                  pl.BlockSpec((1, f_out), lambda i: (0, 0)),
                  pl.BlockSpec((f_out, 128), lambda i: (0, 0))],
        out_specs=pl.BlockSpec((_BR, 128), lambda i: (i, 0)),
        out_shape=jax.ShapeDtypeStruct((_NPAD, 128), jnp.float32),
    )(s, y, b1, W2, b2, W1n)


def _tc_final(s, y, b1, W2, b2):
    def body(s_ref, y_ref, b1_ref, w2_ref, b2_ref, o_ref):
        agg = s_ref[...] - y_ref[...][:, :_H] + b1_ref[...]
        agg = jnp.where(jnp.isfinite(agg), agg, 0.0)
        z = jnp.dot(agg, w2_ref[...],
                    preferred_element_type=jnp.float32) + b2_ref[...]
        o_ref[...] = jax.nn.sigmoid(z)
    return pl.pallas_call(
        body,
        grid=(_GRID,),
        in_specs=[pl.BlockSpec((_BR, _H), lambda i: (i, 0)),
                  pl.BlockSpec((_BR, 128), lambda i: (i, 0)),
                  pl.BlockSpec((1, _H), lambda i: (0, 0)),
                  pl.BlockSpec((_H, 1), lambda i: (0, 0)),
                  pl.BlockSpec((1, 1), lambda i: (0, 0))],
        out_specs=pl.BlockSpec((_BR, 1), lambda i: (i, 0)),
        out_shape=jax.ShapeDtypeStruct((_NPAD, 1), jnp.float32),
    )(s, y, b1, W2, b2)


def kernel(x, edge_index, W1_1, b1_1, W2_1, b2_1, W1_2, b1_2, W2_2, b2_2,
           W1_3, b1_3, W2_3, b2_3):
    src = edge_index[0].astype(jnp.int32)
    dst = edge_index[1].astype(jnp.int32)

    counts = _k1_hist(dst)
    bedge, meta = _k2_scatter(src, dst, counts)

    xp = jnp.pad(x, ((0, _NPAD - _N), (0, 1)))
    W1_1p = jnp.pad(W1_1, ((0, 1), (0, 64)))
    W1_2p = jnp.pad(W1_2, ((0, 0), (0, 64)))
    W1_3p = jnp.pad(W1_3, ((0, 0), (0, 64)))
    y = _tc_y0(xp, W1_1p)

    s = _k3_segmax(y, meta, bedge).reshape(_NPAD, _H)
    y = _tc_mid(s, y, b1_1.reshape(1, -1), W2_1, b2_1.reshape(1, -1), W1_2p)
    s = _k3_segmax(y, meta, bedge).reshape(_NPAD, _H)
    y = _tc_mid(s, y, b1_2.reshape(1, -1), W2_2, b2_2.reshape(1, -1), W1_3p)
    s = _k3_segmax(y, meta, bedge).reshape(_NPAD, _H)
    out = _tc_final(s, y, b1_3.reshape(1, -1), W2_3, b2_3.reshape(1, -1))
    return out[:_N]
